# baseline (speedup 1.0000x reference)
import functools
import numpy as np
import jax
import jax.numpy as jnp
from jax import lax
from jax.experimental import pallas as pl
from jax.experimental.pallas import tpu as pltpu
from jax.experimental.pallas import tpu_sc as plsc

LQ = 577
MRP = 14
NU = 64
NT = 2 * MRP + 2

NC, NS, L = 2, 16, 16
NW = NC * NS

TBLN = 24 * 29 * 24 + 1
R0 = TBLN - 1
TBLP = -(-TBLN // (8 * NW)) * 8 * NW
RPW = TBLP // NW


def _combine_body(tv_ref, th_ref, s_ref):
    s_ref[...] = tv_ref[...][:, None, :] + th_ref[...][None, :, :]


def _make_combined(tv, th):
    out3 = pl.pallas_call(
        _combine_body,
        out_shape=jax.ShapeDtypeStruct((NT, NT, NU), jnp.float32),
    )(tv, th)
    return out3.reshape(NT * NT, NU)


def _tbl_indices():
    q = np.arange(24)[:, None, None]
    d = np.arange(29)[None, :, None]
    r = np.arange(24)[None, None, :]
    fv = d + 1 + np.zeros_like(q) + np.zeros_like(r)
    fh = np.clip(r - q, -MRP, MRP) + MRP + 1 + np.zeros_like(d)
    idx = (fv * NT + fh).reshape(-1)
    idx = np.concatenate([idx, [0]])
    pad = np.zeros((TBLP,), np.int32)
    pad[:TBLN] = idx
    return pad


def _sc_body(s_hbm, tidx_hbm, tbl_hbm, idx_ref, rows_ref, s_shared, gsem):
    wid = lax.axis_index("s") * NC + lax.axis_index("c")

    @pl.when(lax.axis_index("s") == 0)
    def _stage():
        pltpu.sync_copy(s_hbm, s_shared)
    plsc.subcore_barrier()

    base = wid * RPW
    pltpu.sync_copy(tidx_hbm.at[pl.ds(base, RPW)], idx_ref)
    pltpu.async_copy(s_shared.at[idx_ref], rows_ref, gsem).wait()
    pltpu.sync_copy(rows_ref, tbl_hbm.at[pl.ds(base, RPW)])


def _build_block_table(s):
    mesh = plsc.VectorSubcoreMesh(core_axis_name="c", subcore_axis_name="s")
    return pl.kernel(
        _sc_body,
        out_type=jax.ShapeDtypeStruct((TBLP, NU), jnp.float32),
        mesh=mesh,
        compiler_params=pltpu.CompilerParams(use_tc_tiling_on_sc=False),
        scratch_types=[
            pltpu.VMEM((RPW,), jnp.int32),
            pltpu.VMEM((RPW, NU), jnp.float32),
            pltpu.VMEM_SHARED((NT * NT, NU), jnp.float32),
            pltpu.SemaphoreType.DMA,
        ],
    )(s, jnp.asarray(_tbl_indices()))


def _asm_body(tbl_ref, out_ref):
    i = pl.program_id(0)

    @pl.when(i == 0)
    def _row0():
        out_ref[...] = jnp.broadcast_to(tbl_ref[R0][None, None, :],
                                        (1, LQ, NU))

    @pl.when(i > 0)
    def _row():
        im1 = i - 1
        qiv = lax.div(im1, 24)
        qih = im1 - 24 * qiv
        out_ref[0, pl.ds(0, 1), :] = tbl_ref[pl.ds(R0, 1), :]
        for b in range(24):
            dvb = jnp.clip(b - qiv, -MRP, MRP) + MRP
            off = (qih * 29 + dvb) * 24
            out_ref[0, pl.ds(1 + 24 * b, 24), :] = tbl_ref[pl.ds(off, 24), :]


def _assemble(tbl):
    return pl.pallas_call(
        _asm_body,
        grid=(LQ,),
        in_specs=[pl.BlockSpec((TBLP, NU), lambda g: (0, 0))],
        out_specs=pl.BlockSpec((1, LQ, NU), lambda g: (g, 0, 0)),
        out_shape=jax.ShapeDtypeStruct((LQ, LQ, NU), jnp.float32),
    )(tbl)


def kernel(length_q, length_k, sample_embeddings_table_v, sample_embeddings_table_h):
    zero = (length_q - LQ) + (length_k - LQ)
    shift = jnp.clip(jnp.arange(NT) + zero, 0, NT - 1)
    tv = jnp.take(sample_embeddings_table_v, shift, axis=0)
    th = jnp.take(sample_embeddings_table_h, shift, axis=0)
    s = _make_combined(tv, th)
    tbl = _build_block_table(s)
    return _assemble(tbl)


# speedup vs baseline: 1.6373x; 1.6373x over previous
import functools
import numpy as np
import jax
import jax.numpy as jnp
from jax import lax
from jax.experimental import pallas as pl
from jax.experimental.pallas import tpu as pltpu
from jax.experimental.pallas import tpu_sc as plsc

LQ = 577
MRP = 14
NU = 64
NT = 2 * MRP + 2

NC, NS, L = 2, 16, 16
NW = NC * NS

TBLN = 24 * 29 * 24 + 1
R0 = TBLN - 1
TBLP = -(-TBLN // (8 * NW)) * 8 * NW
RPW = TBLP // NW


def _combine_body(tv_ref, th_ref, s_ref):
    s_ref[...] = tv_ref[...][:, None, :] + th_ref[...][None, :, :]


def _make_combined(tv, th):
    out3 = pl.pallas_call(
        _combine_body,
        out_shape=jax.ShapeDtypeStruct((NT, NT, NU), jnp.float32),
    )(tv, th)
    return out3.reshape(NT * NT, NU)


def _tbl_indices():
    q = np.arange(24)[:, None, None]
    d = np.arange(29)[None, :, None]
    r = np.arange(24)[None, None, :]
    fv = d + 1 + np.zeros_like(q) + np.zeros_like(r)
    fh = np.clip(r - q, -MRP, MRP) + MRP + 1 + np.zeros_like(d)
    idx = (fv * NT + fh).reshape(-1)
    idx = np.concatenate([idx, [0]])
    pad = np.zeros((TBLP,), np.int32)
    pad[:TBLN] = idx
    return pad


def _sc_body(s_hbm, tidx_hbm, tbl_hbm, idx_ref, rows_ref, s_shared, gsem):
    wid = lax.axis_index("s") * NC + lax.axis_index("c")

    @pl.when(lax.axis_index("s") == 0)
    def _stage():
        pltpu.sync_copy(s_hbm, s_shared)
    plsc.subcore_barrier()

    base = wid * RPW
    pltpu.sync_copy(tidx_hbm.at[pl.ds(base, RPW)], idx_ref)
    pltpu.async_copy(s_shared.at[idx_ref], rows_ref, gsem).wait()
    pltpu.sync_copy(rows_ref, tbl_hbm.at[pl.ds(base, RPW)])


def _build_block_table(s):
    mesh = plsc.VectorSubcoreMesh(core_axis_name="c", subcore_axis_name="s")
    return pl.kernel(
        _sc_body,
        out_type=jax.ShapeDtypeStruct((TBLP, NU), jnp.float32),
        mesh=mesh,
        compiler_params=pltpu.CompilerParams(use_tc_tiling_on_sc=False),
        scratch_types=[
            pltpu.VMEM((RPW,), jnp.int32),
            pltpu.VMEM((RPW, NU), jnp.float32),
            pltpu.VMEM_SHARED((NT * NT, NU), jnp.float32),
            pltpu.SemaphoreType.DMA,
        ],
    )(s, jnp.asarray(_tbl_indices()))


IB = 8


def _asm_body(tbl_ref, out_ref):
    g = pl.program_id(0)
    for ii in range(IB):
        i = g * IB + ii

        @pl.when(i == 0)
        def _row0(ii=ii):
            out_ref[ii, :, :] = jnp.broadcast_to(tbl_ref[R0][None, :],
                                                 (LQ, NU))

        @pl.when((i > 0) & (i < LQ))
        def _row(i=i, ii=ii):
            im1 = i - 1
            qiv = lax.div(im1, 24)
            qih = im1 - 24 * qiv
            out_ref[ii, pl.ds(0, 1), :] = tbl_ref[pl.ds(R0, 1), :]
            for b in range(24):
                dvb = jnp.clip(b - qiv, -MRP, MRP) + MRP
                off = (qih * 29 + dvb) * 24
                out_ref[ii, pl.ds(1 + 24 * b, 24), :] = tbl_ref[pl.ds(off, 24), :]


def _assemble(tbl):
    return pl.pallas_call(
        _asm_body,
        grid=(-(-LQ // IB),),
        in_specs=[pl.BlockSpec((TBLP, NU), lambda g: (0, 0))],
        out_specs=pl.BlockSpec((IB, LQ, NU), lambda g: (g, 0, 0)),
        out_shape=jax.ShapeDtypeStruct((LQ, LQ, NU), jnp.float32),
    )(tbl)


def kernel(length_q, length_k, sample_embeddings_table_v, sample_embeddings_table_h):
    zero = (length_q - LQ) + (length_k - LQ)
    shift = jnp.clip(jnp.arange(NT) + zero, 0, NT - 1)
    tv = jnp.take(sample_embeddings_table_v, shift, axis=0)
    th = jnp.take(sample_embeddings_table_h, shift, axis=0)
    s = _make_combined(tv, th)
    tbl = _build_block_table(s)
    return _assemble(tbl)


# speedup vs baseline: 1.7145x; 1.0471x over previous
import functools
import numpy as np
import jax
import jax.numpy as jnp
from jax import lax
from jax.experimental import pallas as pl
from jax.experimental.pallas import tpu as pltpu
from jax.experimental.pallas import tpu_sc as plsc

LQ = 577
MRP = 14
NU = 64
NT = 2 * MRP + 2

NC, NS, L = 2, 16, 16
NW = NC * NS

TBLN = 24 * 29 * 24 + 1
R0 = TBLN - 1
TBLP = -(-TBLN // (8 * NW)) * 8 * NW
RPW = TBLP // NW


def _combine_body(tv_ref, th_ref, s_ref):
    s_ref[...] = tv_ref[...][:, None, :] + th_ref[...][None, :, :]


def _make_combined(tv, th):
    out3 = pl.pallas_call(
        _combine_body,
        out_shape=jax.ShapeDtypeStruct((NT, NT, NU), jnp.float32),
    )(tv, th)
    return out3.reshape(NT * NT, NU)


def _tbl_indices():
    q = np.arange(24)[:, None, None]
    d = np.arange(29)[None, :, None]
    r = np.arange(24)[None, None, :]
    fv = d + 1 + np.zeros_like(q) + np.zeros_like(r)
    fh = np.clip(r - q, -MRP, MRP) + MRP + 1 + np.zeros_like(d)
    idx = (fv * NT + fh).reshape(-1)
    idx = np.concatenate([idx, [0]])
    pad = np.zeros((TBLP,), np.int32)
    pad[:TBLN] = idx
    return pad


def _sc_body(s_hbm, tidx_hbm, tbl_hbm, idx_ref, rows_ref, s_shared, gsem):
    wid = lax.axis_index("s") * NC + lax.axis_index("c")

    @pl.when(lax.axis_index("s") == 0)
    def _stage():
        pltpu.sync_copy(s_hbm, s_shared)
    plsc.subcore_barrier()

    base = wid * RPW
    pltpu.sync_copy(tidx_hbm.at[pl.ds(base, RPW)], idx_ref)
    pltpu.async_copy(s_shared.at[idx_ref], rows_ref, gsem).wait()
    pltpu.sync_copy(rows_ref, tbl_hbm.at[pl.ds(base, RPW)])


def _build_block_table(s):
    mesh = plsc.VectorSubcoreMesh(core_axis_name="c", subcore_axis_name="s")
    return pl.kernel(
        _sc_body,
        out_type=jax.ShapeDtypeStruct((TBLP, NU), jnp.float32),
        mesh=mesh,
        compiler_params=pltpu.CompilerParams(use_tc_tiling_on_sc=False),
        scratch_types=[
            pltpu.VMEM((RPW,), jnp.int32),
            pltpu.VMEM((RPW, NU), jnp.float32),
            pltpu.VMEM_SHARED((NT * NT, NU), jnp.float32),
            pltpu.SemaphoreType.DMA,
        ],
    )(s, jnp.asarray(_tbl_indices()))


IB = 16


def _asm_body(tbl_ref, out_ref):
    g = pl.program_id(0)
    for ii in range(IB):
        i = g * IB + ii

        @pl.when(i == 0)
        def _row0(ii=ii):
            out_ref[ii, :, :] = jnp.broadcast_to(tbl_ref[R0][None, :],
                                                 (LQ, NU))

        @pl.when((i > 0) & (i < LQ))
        def _row(i=i, ii=ii):
            im1 = i - 1
            qiv = lax.div(im1, 24)
            qih = im1 - 24 * qiv
            out_ref[ii, pl.ds(0, 1), :] = tbl_ref[pl.ds(R0, 1), :]
            for b in range(24):
                dvb = jnp.clip(b - qiv, -MRP, MRP) + MRP
                off = (qih * 29 + dvb) * 24
                out_ref[ii, pl.ds(1 + 24 * b, 24), :] = tbl_ref[pl.ds(off, 24), :]


def _assemble(tbl):
    return pl.pallas_call(
        _asm_body,
        grid=(-(-LQ // IB),),
        in_specs=[pl.BlockSpec((TBLP, NU), lambda g: (0, 0))],
        out_specs=pl.BlockSpec((IB, LQ, NU), lambda g: (g, 0, 0)),
        out_shape=jax.ShapeDtypeStruct((LQ, LQ, NU), jnp.float32),
    )(tbl)


def kernel(length_q, length_k, sample_embeddings_table_v, sample_embeddings_table_h):
    zero = (length_q - LQ) + (length_k - LQ)
    shift = jnp.clip(jnp.arange(NT) + zero, 0, NT - 1)
    tv = jnp.take(sample_embeddings_table_v, shift, axis=0)
    th = jnp.take(sample_embeddings_table_h, shift, axis=0)
    s = _make_combined(tv, th)
    tbl = _build_block_table(s)
    return _assemble(tbl)


# speedup vs baseline: 1.7472x; 1.0191x over previous
import functools
import numpy as np
import jax
import jax.numpy as jnp
from jax import lax
from jax.experimental import pallas as pl
from jax.experimental.pallas import tpu as pltpu
from jax.experimental.pallas import tpu_sc as plsc

LQ = 577
MRP = 14
NU = 64
NT = 2 * MRP + 2

NC, NS, L = 2, 16, 16
NW = NC * NS

TBLN = 24 * 29 * 24 + 1
R0 = TBLN - 1
TBLP = -(-TBLN // (8 * NW)) * 8 * NW
RPW = TBLP // NW


def _combine_body(tv_ref, th_ref, s_ref):
    s_ref[...] = tv_ref[...][:, None, :] + th_ref[...][None, :, :]


def _make_combined(tv, th):
    out3 = pl.pallas_call(
        _combine_body,
        out_shape=jax.ShapeDtypeStruct((NT, NT, NU), jnp.float32),
    )(tv, th)
    return out3.reshape(NT * NT, NU)


def _tbl_indices():
    q = np.arange(24)[:, None, None]
    d = np.arange(29)[None, :, None]
    r = np.arange(24)[None, None, :]
    fv = d + 1 + np.zeros_like(q) + np.zeros_like(r)
    fh = np.clip(r - q, -MRP, MRP) + MRP + 1 + np.zeros_like(d)
    idx = (fv * NT + fh).reshape(-1)
    idx = np.concatenate([idx, [0]])
    pad = np.zeros((TBLP,), np.int32)
    pad[:TBLN] = idx
    return pad


def _sc_body(s_hbm, tidx_hbm, tbl_hbm, idx_ref, rows_ref, s_shared, gsem):
    wid = lax.axis_index("s") * NC + lax.axis_index("c")

    @pl.when(lax.axis_index("s") == 0)
    def _stage():
        pltpu.sync_copy(s_hbm, s_shared)
    plsc.subcore_barrier()

    base = wid * RPW
    pltpu.sync_copy(tidx_hbm.at[pl.ds(base, RPW)], idx_ref)
    pltpu.async_copy(s_shared.at[idx_ref], rows_ref, gsem).wait()
    pltpu.sync_copy(rows_ref, tbl_hbm.at[pl.ds(base, RPW)])


def _build_block_table(s):
    mesh = plsc.VectorSubcoreMesh(core_axis_name="c", subcore_axis_name="s")
    return pl.kernel(
        _sc_body,
        out_type=jax.ShapeDtypeStruct((TBLP, NU), jnp.float32),
        mesh=mesh,
        compiler_params=pltpu.CompilerParams(use_tc_tiling_on_sc=False),
        scratch_types=[
            pltpu.VMEM((RPW,), jnp.int32),
            pltpu.VMEM((RPW, NU), jnp.float32),
            pltpu.VMEM_SHARED((NT * NT, NU), jnp.float32),
            pltpu.SemaphoreType.DMA,
        ],
    )(s, jnp.asarray(_tbl_indices()))


IB = 32


def _asm_body(tbl_ref, out_ref):
    g = pl.program_id(0)
    for ii in range(IB):
        i = g * IB + ii

        @pl.when(i == 0)
        def _row0(ii=ii):
            out_ref[ii, :, :] = jnp.broadcast_to(tbl_ref[R0][None, :],
                                                 (LQ, NU))

        @pl.when((i > 0) & (i < LQ))
        def _row(i=i, ii=ii):
            im1 = i - 1
            qiv = lax.div(im1, 24)
            qih = im1 - 24 * qiv
            out_ref[ii, pl.ds(0, 1), :] = tbl_ref[pl.ds(R0, 1), :]
            for b in range(24):
                dvb = jnp.clip(b - qiv, -MRP, MRP) + MRP
                off = (qih * 29 + dvb) * 24
                out_ref[ii, pl.ds(1 + 24 * b, 24), :] = tbl_ref[pl.ds(off, 24), :]


def _assemble(tbl):
    return pl.pallas_call(
        _asm_body,
        grid=(-(-LQ // IB),),
        in_specs=[pl.BlockSpec((TBLP, NU), lambda g: (0, 0))],
        out_specs=pl.BlockSpec((IB, LQ, NU), lambda g: (g, 0, 0)),
        out_shape=jax.ShapeDtypeStruct((LQ, LQ, NU), jnp.float32),
    )(tbl)


def kernel(length_q, length_k, sample_embeddings_table_v, sample_embeddings_table_h):
    zero = (length_q - LQ) + (length_k - LQ)
    shift = jnp.clip(jnp.arange(NT) + zero, 0, NT - 1)
    tv = jnp.take(sample_embeddings_table_v, shift, axis=0)
    th = jnp.take(sample_embeddings_table_h, shift, axis=0)
    s = _make_combined(tv, th)
    tbl = _build_block_table(s)
    return _assemble(tbl)
